# baseline (speedup 1.0000x reference)
import functools

import jax
import jax.numpy as jnp
from jax import lax
from jax.experimental import pallas as pl
from jax.experimental.pallas import tpu as pltpu
from jax.experimental.pallas import tpu_sc as plsc

_S = 256
_KK = 3
_BIG = 1e30
_NEG = -3e38


def _tc_body(x_ref, xs_ref, rand_ref, w_ref, b_ref, tkb_ref, proj_ref, *, T):
    b = pl.program_id(0)
    xt = x_ref[0]
    xs = xs_ref[0]
    norm = jnp.sqrt(jnp.sum(xs * xs, axis=0, keepdims=True))
    sn = xs / (norm + 1e-8)
    norm_t = jnp.sqrt(jnp.sum(xt * xt, axis=0, keepdims=True))
    xn = xt / (norm_t + 1e-8)
    scores = lax.dot_general(sn, xn, (((0,), (0,)), ((), ())))
    tok_id = (lax.broadcasted_iota(jnp.int32, (_S, T), 1)
              + pl.program_id(1) * T)
    scores = jnp.where(tok_id == rand_ref[...], _BIG, scores)
    iota_row = lax.broadcasted_iota(
        jnp.int32, (1, _S), 1).astype(jnp.float32)
    for k in range(_KK):
        m = jnp.max(scores, axis=0, keepdims=True)
        oh_b = scores == m
        if k < _KK - 1:
            scores = jnp.where(oh_b, _NEG, scores)
        a_f = lax.dot_general(
            iota_row, oh_b.astype(jnp.float32),
            (((1,), (0,)), ((), ())),
            precision=lax.Precision.HIGHEST)
        base = (a_f + (b * _KK + k).astype(jnp.float32) * _S) * 32.0
        tkb_ref[0, k:k + 1, :] = base.astype(jnp.int32)
        projk = lax.dot_general(
            xs, w_ref[k], (((0,), (0,)), ((), ())),
            precision=lax.Precision.HIGHEST)
        if k == 0:
            projk = projk + b_ref[...]
        proj_ref[0, k] = projk


def _sc_kernel(tkb_hbm, proj_hbm, out_hbm, idx_v, proj_v, out_v,
               *, N, B, O, NW, CHUNK):
    info = plsc.get_sparse_core_info()
    nc = info.num_cores
    wid = lax.axis_index("s") * nc + lax.axis_index("c")

    @pl.when(wid < NW)
    def _():
        pltpu.sync_copy(proj_hbm, proj_v)
        ngrp = CHUNK // 16
        for b in range(B):
            for k in range(_KK):
                pltpu.sync_copy(
                    tkb_hbm.at[pl.ds((b * _KK + k) * N + wid * CHUNK, CHUNK)],
                    idx_v.at[pl.ds(k * CHUNK, CHUNK)])

            def body(g, carry, b=b):
                a0 = idx_v[pl.ds(0 * CHUNK + g * 16, 16)]
                a1 = idx_v[pl.ds(1 * CHUNK + g * 16, 16)]
                a2 = idx_v[pl.ds(2 * CHUNK + g * 16, 16)]
                for c in range(O):
                    acc = (plsc.load_gather(proj_v, [a0 + c])
                           + plsc.load_gather(proj_v, [a1 + c])
                           + plsc.load_gather(proj_v, [a2 + c]))
                    out_v[c, pl.ds(g * 16, 16)] = acc
                return carry

            lax.fori_loop(0, ngrp, body, 0)
            pltpu.sync_copy(out_v,
                            out_hbm.at[b, :, pl.ds(wid * CHUNK, CHUNK)])


def kernel(x, conv_w, conv_b):
    B, C, H, W = x.shape
    N = H * W
    O = conv_w.shape[0]
    T = 7168
    NT = N // T
    x_flat = x.reshape(B, C, N)
    rand_idx = jax.random.permutation(jax.random.key(42), N)[:_S]
    x_sample = jnp.take(x_flat, rand_idx, axis=2)
    rand_col = rand_idx.astype(jnp.int32).reshape(_S, 1)
    w_r = jnp.transpose(conv_w, (2, 1, 0))
    b_r = conv_b.reshape(1, O)

    tkb, proj = pl.pallas_call(
        functools.partial(_tc_body, T=T),
        grid=(B, NT),
        in_specs=[
            pl.BlockSpec((1, C, T), lambda b, t: (b, 0, t)),
            pl.BlockSpec((1, C, _S), lambda b, t: (b, 0, 0)),
            pl.BlockSpec((_S, 1), lambda b, t: (0, 0)),
            pl.BlockSpec((_KK, C, O), lambda b, t: (0, 0, 0)),
            pl.BlockSpec((1, O), lambda b, t: (0, 0)),
        ],
        out_specs=[
            pl.BlockSpec((1, _KK, T), lambda b, t: (b, 0, t)),
            pl.BlockSpec((1, _KK, _S, O), lambda b, t: (b, 0, 0, 0)),
        ],
        out_shape=[
            jax.ShapeDtypeStruct((B, _KK, N), jnp.int32),
            jax.ShapeDtypeStruct((B, _KK, _S, O), jnp.float32),
        ],
    )(x_flat, x_sample, rand_col, w_r, b_r)

    mesh = plsc.VectorSubcoreMesh(core_axis_name="c", subcore_axis_name="s")
    nw = 28
    chunk = N // nw
    sc = functools.partial(
        pl.kernel,
        mesh=mesh,
        compiler_params=pltpu.CompilerParams(needs_layout_passes=False),
        out_type=jax.ShapeDtypeStruct((B, O, N), jnp.float32),
        scratch_types=[
            pltpu.VMEM((_KK * chunk,), jnp.int32),
            pltpu.VMEM((B * _KK * _S * O,), jnp.float32),
            pltpu.VMEM((O, chunk), jnp.float32),
        ],
    )(functools.partial(_sc_kernel, N=N, B=B, O=O, NW=nw, CHUNK=chunk))
    out = sc(tkb.reshape(-1), proj.reshape(-1))
    return out.reshape(B, O, H, W)


# speedup vs baseline: 1.0940x; 1.0940x over previous
import functools

import jax
import jax.numpy as jnp
from jax import lax
from jax.experimental import pallas as pl
from jax.experimental.pallas import tpu as pltpu
from jax.experimental.pallas import tpu_sc as plsc

_S = 256
_KK = 3
_BIG = 1e30
_NEG = -3e38


def _tc_body(x_ref, xs_ref, rand_ref, w_ref, b_ref, tkb_ref, proj_ref, *, T):
    b = pl.program_id(0)
    xt = x_ref[0]
    xs = xs_ref[0]
    norm = jnp.sqrt(jnp.sum(xs * xs, axis=0, keepdims=True))
    sn = xs / (norm + 1e-8)
    norm_t = jnp.sqrt(jnp.sum(xt * xt, axis=0, keepdims=True))
    xn = xt / (norm_t + 1e-8)
    scores = lax.dot_general(sn, xn, (((0,), (0,)), ((), ())))
    tok_id = (lax.broadcasted_iota(jnp.int32, (_S, T), 1)
              + pl.program_id(1) * T)
    scores = jnp.where(tok_id == rand_ref[...], _BIG, scores)
    iota_row = lax.broadcasted_iota(
        jnp.int32, (1, _S), 1).astype(jnp.float32)
    for k in range(_KK):
        m = jnp.max(scores, axis=0, keepdims=True)
        oh_b = scores == m
        if k < _KK - 1:
            scores = jnp.where(oh_b, _NEG, scores)
        a_f = lax.dot_general(
            iota_row, oh_b.astype(jnp.float32),
            (((1,), (0,)), ((), ())),
            precision=lax.Precision.HIGHEST)
        base = (a_f + (b * _KK + k).astype(jnp.float32) * _S) * 32.0
        tkb_ref[0, k:k + 1, :] = base.astype(jnp.int32)
        projk = lax.dot_general(
            xs, w_ref[k], (((0,), (0,)), ((), ())),
            precision=lax.Precision.HIGHEST)
        if k == 0:
            projk = projk + b_ref[...]
        proj_ref[0, k] = projk


def _sc_kernel(tkb_hbm, proj_hbm, out_hbm, idx_v, proj_v, out_v,
               *, N, B, O, NW, CHUNK):
    info = plsc.get_sparse_core_info()
    nc = info.num_cores
    wid = lax.axis_index("s") * nc + lax.axis_index("c")

    @pl.when(wid < NW)
    def _():
        pltpu.sync_copy(proj_hbm, proj_v)
        ngrp = CHUNK // 16
        for b in range(B):
            for k in range(_KK):
                pltpu.sync_copy(
                    tkb_hbm.at[pl.ds((b * _KK + k) * N + wid * CHUNK, CHUNK)],
                    idx_v.at[pl.ds(k * CHUNK, CHUNK)])

            @plsc.parallel_loop(0, ngrp, unroll=2)
            def body(g, b=b):
                a0 = idx_v[pl.ds(0 * CHUNK + g * 16, 16)]
                a1 = idx_v[pl.ds(1 * CHUNK + g * 16, 16)]
                a2 = idx_v[pl.ds(2 * CHUNK + g * 16, 16)]
                for c in range(O):
                    acc = (plsc.load_gather(proj_v, [a0 + c])
                           + plsc.load_gather(proj_v, [a1 + c])
                           + plsc.load_gather(proj_v, [a2 + c]))
                    out_v[c, pl.ds(g * 16, 16)] = acc

            pltpu.sync_copy(out_v,
                            out_hbm.at[b, :, pl.ds(wid * CHUNK, CHUNK)])


def kernel(x, conv_w, conv_b):
    B, C, H, W = x.shape
    N = H * W
    O = conv_w.shape[0]
    T = 7168
    NT = N // T
    x_flat = x.reshape(B, C, N)
    rand_idx = jax.random.permutation(jax.random.key(42), N)[:_S]
    x_sample = jnp.take(x_flat, rand_idx, axis=2)
    rand_col = rand_idx.astype(jnp.int32).reshape(_S, 1)
    w_r = jnp.transpose(conv_w, (2, 1, 0))
    b_r = conv_b.reshape(1, O)

    tkb, proj = pl.pallas_call(
        functools.partial(_tc_body, T=T),
        grid=(B, NT),
        in_specs=[
            pl.BlockSpec((1, C, T), lambda b, t: (b, 0, t)),
            pl.BlockSpec((1, C, _S), lambda b, t: (b, 0, 0)),
            pl.BlockSpec((_S, 1), lambda b, t: (0, 0)),
            pl.BlockSpec((_KK, C, O), lambda b, t: (0, 0, 0)),
            pl.BlockSpec((1, O), lambda b, t: (0, 0)),
        ],
        out_specs=[
            pl.BlockSpec((1, _KK, T), lambda b, t: (b, 0, t)),
            pl.BlockSpec((1, _KK, _S, O), lambda b, t: (b, 0, 0, 0)),
        ],
        out_shape=[
            jax.ShapeDtypeStruct((B, _KK, N), jnp.int32),
            jax.ShapeDtypeStruct((B, _KK, _S, O), jnp.float32),
        ],
    )(x_flat, x_sample, rand_col, w_r, b_r)

    mesh = plsc.VectorSubcoreMesh(core_axis_name="c", subcore_axis_name="s")
    nw = 28
    chunk = N // nw
    sc = functools.partial(
        pl.kernel,
        mesh=mesh,
        compiler_params=pltpu.CompilerParams(needs_layout_passes=False),
        out_type=jax.ShapeDtypeStruct((B, O, N), jnp.float32),
        scratch_types=[
            pltpu.VMEM((_KK * chunk,), jnp.int32),
            pltpu.VMEM((B * _KK * _S * O,), jnp.float32),
            pltpu.VMEM((O, chunk), jnp.float32),
        ],
    )(functools.partial(_sc_kernel, N=N, B=B, O=O, NW=nw, CHUNK=chunk))
    out = sc(tkb.reshape(-1), proj.reshape(-1))
    return out.reshape(B, O, H, W)


# speedup vs baseline: 1.2836x; 1.1733x over previous
import functools

import jax
import jax.numpy as jnp
import numpy as np
from jax import lax
from jax.experimental import pallas as pl
from jax.experimental.pallas import tpu as pltpu
from jax.experimental.pallas import tpu_sc as plsc

_S = 256
_KK = 3
_BIG = 1e30
_NEG = -3e38

_N_TOKENS = 224 * 224
with jax.default_device(jax.local_devices(backend="cpu")[0]):
    _RAND_IDX = np.asarray(
        jax.random.permutation(jax.random.key(42), _N_TOKENS)[:_S]
    ).astype(np.int32)


def _tc_body(x_ref, xs_ref, rand_ref, w_ref, b_ref, tkb_ref, proj_ref, *, T):
    b = pl.program_id(0)
    xt = x_ref[0]
    xs = xs_ref[0]
    norm = jnp.sqrt(jnp.sum(xs * xs, axis=0, keepdims=True))
    sn = xs / (norm + 1e-8)
    norm_t = jnp.sqrt(jnp.sum(xt * xt, axis=0, keepdims=True))
    xn = xt / (norm_t + 1e-8)
    scores = lax.dot_general(sn, xn, (((0,), (0,)), ((), ())))
    tok_id = (lax.broadcasted_iota(jnp.int32, (_S, T), 1)
              + pl.program_id(1) * T)
    scores = jnp.where(tok_id == rand_ref[...], _BIG, scores)
    iota_row = lax.broadcasted_iota(
        jnp.int32, (1, _S), 1).astype(jnp.float32)
    for k in range(_KK):
        m = jnp.max(scores, axis=0, keepdims=True)
        oh_b = scores == m
        if k < _KK - 1:
            scores = jnp.where(oh_b, _NEG, scores)
        a_f = lax.dot_general(
            iota_row, oh_b.astype(jnp.float32),
            (((1,), (0,)), ((), ())),
            precision=lax.Precision.HIGHEST)
        base = (a_f + (b * _KK + k).astype(jnp.float32) * _S) * 32.0
        tkb_ref[0, k:k + 1, :] = base.astype(jnp.int32)
        projk = lax.dot_general(
            xs, w_ref[k], (((0,), (0,)), ((), ())),
            precision=lax.Precision.HIGHEST)
        if k == 0:
            projk = projk + b_ref[...]
        proj_ref[0, k] = projk


def _sc_kernel(tkb_hbm, proj_hbm, out_hbm, idx_v, proj_v, out_v,
               *, N, B, O, NW, CHUNK):
    info = plsc.get_sparse_core_info()
    nc = info.num_cores
    wid = lax.axis_index("s") * nc + lax.axis_index("c")

    @pl.when(wid < NW)
    def _():
        pltpu.sync_copy(proj_hbm, proj_v)
        ngrp = CHUNK // 16
        for b in range(B):
            for k in range(_KK):
                pltpu.sync_copy(
                    tkb_hbm.at[pl.ds((b * _KK + k) * N + wid * CHUNK, CHUNK)],
                    idx_v.at[pl.ds(k * CHUNK, CHUNK)])

            @plsc.parallel_loop(0, ngrp, unroll=2)
            def body(g, b=b):
                a0 = idx_v[pl.ds(0 * CHUNK + g * 16, 16)]
                a1 = idx_v[pl.ds(1 * CHUNK + g * 16, 16)]
                a2 = idx_v[pl.ds(2 * CHUNK + g * 16, 16)]
                for c in range(O):
                    acc = (plsc.load_gather(proj_v, [a0 + c])
                           + plsc.load_gather(proj_v, [a1 + c])
                           + plsc.load_gather(proj_v, [a2 + c]))
                    out_v[c, pl.ds(g * 16, 16)] = acc

            pltpu.sync_copy(out_v,
                            out_hbm.at[b, :, pl.ds(wid * CHUNK, CHUNK)])


def kernel(x, conv_w, conv_b):
    B, C, H, W = x.shape
    N = H * W
    O = conv_w.shape[0]
    T = 7168
    NT = N // T
    x_flat = x.reshape(B, C, N)
    x_sample = jnp.take(x_flat, _RAND_IDX, axis=2)
    rand_col = jnp.asarray(_RAND_IDX.reshape(_S, 1))
    w_r = jnp.transpose(conv_w, (2, 1, 0))
    b_r = conv_b.reshape(1, O)

    tkb, proj = pl.pallas_call(
        functools.partial(_tc_body, T=T),
        grid=(B, NT),
        in_specs=[
            pl.BlockSpec((1, C, T), lambda b, t: (b, 0, t)),
            pl.BlockSpec((1, C, _S), lambda b, t: (b, 0, 0)),
            pl.BlockSpec((_S, 1), lambda b, t: (0, 0)),
            pl.BlockSpec((_KK, C, O), lambda b, t: (0, 0, 0)),
            pl.BlockSpec((1, O), lambda b, t: (0, 0)),
        ],
        out_specs=[
            pl.BlockSpec((1, _KK, T), lambda b, t: (b, 0, t)),
            pl.BlockSpec((1, _KK, _S, O), lambda b, t: (b, 0, 0, 0)),
        ],
        out_shape=[
            jax.ShapeDtypeStruct((B, _KK, N), jnp.int32),
            jax.ShapeDtypeStruct((B, _KK, _S, O), jnp.float32),
        ],
    )(x_flat, x_sample, rand_col, w_r, b_r)

    mesh = plsc.VectorSubcoreMesh(core_axis_name="c", subcore_axis_name="s")
    nw = 28
    chunk = N // nw
    sc = functools.partial(
        pl.kernel,
        mesh=mesh,
        compiler_params=pltpu.CompilerParams(needs_layout_passes=False),
        out_type=jax.ShapeDtypeStruct((B, O, N), jnp.float32),
        scratch_types=[
            pltpu.VMEM((_KK * chunk,), jnp.int32),
            pltpu.VMEM((B * _KK * _S * O,), jnp.float32),
            pltpu.VMEM((O, chunk), jnp.float32),
        ],
    )(functools.partial(_sc_kernel, N=N, B=B, O=O, NW=nw, CHUNK=chunk))
    out = sc(tkb.reshape(-1), proj.reshape(-1))
    return out.reshape(B, O, H, W)


# speedup vs baseline: 4.4168x; 3.4410x over previous
import functools

import jax
import jax.numpy as jnp
import numpy as np
from jax import lax
from jax.experimental import pallas as pl

_N_TOKENS = 224 * 224
with jax.default_device(jax.local_devices(backend="cpu")[0]):
    _RAND_IDX = np.asarray(
        jax.random.permutation(jax.random.key(42), _N_TOKENS)[:256]
    ).astype(np.int32)

_S = 256
_KK = 3
_BIG = 1e30
_NEG = -3e38


def _tc_body(x_ref, xs_ref, rand_ref, w_ref, b_ref, out_ref, *, T):
    xt = x_ref[0]
    xs = xs_ref[0]
    norm = jnp.sqrt(jnp.sum(xs * xs, axis=0, keepdims=True))
    sn = xs / (norm + 1e-8)
    norm_t = jnp.sqrt(jnp.sum(xt * xt, axis=0, keepdims=True))
    xn = xt / (norm_t + 1e-8)
    scores = lax.dot_general(sn, xn, (((0,), (0,)), ((), ())))
    tok_id = (lax.broadcasted_iota(jnp.int32, (_S, T), 1)
              + pl.program_id(1) * T)
    scores = jnp.where(tok_id == rand_ref[...], _BIG, scores)
    acc = jnp.broadcast_to(b_ref[...], (32, T))
    for k in range(_KK):
        m = jnp.max(scores, axis=0, keepdims=True)
        oh_b = scores == m
        if k < _KK - 1:
            scores = jnp.where(oh_b, _NEG, scores)
        projk = lax.dot_general(
            xs, w_ref[k], (((0,), (0,)), ((), ())),
            precision=lax.Precision.HIGHEST)
        acc = acc + lax.dot_general(
            projk, oh_b.astype(jnp.float32),
            (((0,), (0,)), ((), ())))
    out_ref[0] = acc


def kernel(x, conv_w, conv_b):
    B, C, H, W = x.shape
    N = H * W
    O = conv_w.shape[0]
    T = 7168
    NT = N // T
    x_flat = x.reshape(B, C, N)
    x_sample = jnp.take(x_flat, _RAND_IDX, axis=2)
    rand_col = jnp.asarray(_RAND_IDX.reshape(_S, 1))
    w_r = jnp.transpose(conv_w, (2, 1, 0))
    b_r = conv_b.reshape(O, 1)

    out = pl.pallas_call(
        functools.partial(_tc_body, T=T),
        grid=(B, NT),
        in_specs=[
            pl.BlockSpec((1, C, T), lambda b, t: (b, 0, t)),
            pl.BlockSpec((1, C, _S), lambda b, t: (b, 0, 0)),
            pl.BlockSpec((_S, 1), lambda b, t: (0, 0)),
            pl.BlockSpec((_KK, C, O), lambda b, t: (0, 0, 0)),
            pl.BlockSpec((O, 1), lambda b, t: (0, 0)),
        ],
        out_specs=pl.BlockSpec((1, O, T), lambda b, t: (b, 0, t)),
        out_shape=jax.ShapeDtypeStruct((B, O, N), jnp.float32),
    )(x_flat, x_sample, rand_col, w_r, b_r)
    return out.reshape(B, O, H, W)
